# baseline (speedup 1.0000x reference)
import functools

import jax
import jax.numpy as jnp
from jax.experimental import pallas as pl

U = 50000; I = 50000; G = 4; D = 128; H = 4; DK = 32
LGNN = 2; LATT = 2; P = 50; B = 4096; NNZ = 500000; LEAKY = 0.5


def _lrelu(x):
    return jnp.where(x > 0, x, LEAKY * x)


def _ln(x, g, b, eps=1e-5):
    m = x.mean(-1, keepdims=True)
    v = ((x - m) ** 2).mean(-1, keepdims=True)
    return (x - m) / jnp.sqrt(v + eps) * g + b


def _mhsa(x, Wq, Wk, Wv):
    Bx, S, Dm = x.shape
    q = (x @ Wq.T).reshape(Bx, S, H, DK).transpose(0, 2, 1, 3)
    k = (x @ Wk.T).reshape(Bx, S, H, DK).transpose(0, 2, 1, 3)
    v = (x @ Wv.T).reshape(Bx, S, H, DK).transpose(0, 2, 1, 3)
    scores = q @ k.transpose(0, 1, 3, 2) / DK ** 0.5
    attn = jax.nn.softmax(scores, axis=-1)
    ctx = (attn @ v).transpose(0, 2, 1, 3).reshape(Bx, S, Dm)
    return ctx


def _lstm(x, Wih, Whh, bih, bhh):
    N = x.shape[0]
    def step(carry, xt):
        h, c = carry
        gates = xt @ Wih.T + h @ Whh.T + bih + bhh
        i, f, g, o = jnp.split(gates, 4, axis=-1)
        i = jax.nn.sigmoid(i); f = jax.nn.sigmoid(f)
        g = jnp.tanh(g); o = jax.nn.sigmoid(o)
        c = f * c + i * g
        h = o * jnp.tanh(c)
        return (h, c), h
    h0 = jnp.zeros((N, D), x.dtype); c0 = jnp.zeros((N, D), x.dtype)
    _, hs = jax.lax.scan(step, (h0, c0), x.transpose(1, 0, 2))
    return hs.transpose(1, 0, 2)


def _spmm(vals, rows, cols, x, n_out):
    return jax.ops.segment_sum(vals[:, None] * x[cols], rows, num_segments=n_out)


_ROWS_BLK = 4000


def _epilogue_body(s_ref, cur_ref, acc_ref, cur_out_ref, acc_out_ref):
    s = s_ref[...]
    cur_new = jnp.where(s > 0, s, LEAKY * s) + cur_ref[...]
    cur_out_ref[...] = cur_new
    acc_out_ref[...] = acc_ref[...] + cur_new


def _gnn_epilogue(s, cur, acc):
    n = s.shape[0] * s.shape[1]
    s2 = s.reshape(n, D); c2 = cur.reshape(n, D); a2 = acc.reshape(n, D)
    grid = n // _ROWS_BLK
    spec = pl.BlockSpec((_ROWS_BLK, D), lambda i: (i, 0))
    cur_new, acc_new = pl.pallas_call(
        _epilogue_body,
        grid=(grid,),
        in_specs=[spec, spec, spec],
        out_specs=[spec, spec],
        out_shape=[jax.ShapeDtypeStruct((n, D), jnp.float32)] * 2,
    )(s2, c2, a2)
    return cur_new.reshape(s.shape), acc_new.reshape(s.shape)


def _forward(params, adj_vals, masks, adj_rows, adj_cols, uids, iids, sequences, u_locs_seq):
    cur = jnp.concatenate([params["user_embeds"], params["item_embeds"]], axis=0)
    acc = cur
    for _ in range(LGNN):
        s_list = []
        for k in range(G):
            rows, cols, vals = adj_rows[k], adj_cols[k], adj_vals[k]
            s_list.append(_spmm(vals, rows, cols, cur[G + k], U))
        for k in range(G):
            rows, cols, vals = adj_rows[k], adj_cols[k], adj_vals[k]
            s_list.append(_spmm(vals, cols, rows, cur[k], I))
        s = jnp.stack(s_list, axis=0)
        cur, acc = _gnn_epilogue(s, cur, acc)
    user_stack = acc[:G].transpose(1, 0, 2)
    item_stack = acc[G:].transpose(1, 0, 2)

    user_rnn = _lstm(user_stack, *params["lstm_user"])
    item_rnn = _lstm(item_stack, *params["lstm_item"])
    user_att = _mhsa(_ln(user_rnn, *params["ln_user"]), *params["mhsa_user"])
    item_att = _mhsa(_ln(item_rnn, *params["ln_item"]), *params["mhsa_item"])
    final_user = user_att.mean(axis=1)
    final_item = item_att.mean(axis=1)

    seq_emb = final_item[sequences]
    pos_emb = jnp.broadcast_to(params["pos_embed"][None], (sequences.shape[0], P, D))
    mask_exp = masks[:, :, None]
    att = (_ln(seq_emb, *params["ln_seq"]) + _ln(pos_emb, *params["ln_seq_pos"])) * mask_exp
    for i in range(LATT):
        att_new = _mhsa(_ln(att, *params["ln_seq_layers"][i]), *params["seq_mhsa"][i])
        att = (_lrelu(att_new) + att) * mask_exp
    att_user = att.sum(axis=1)
    u_emb = final_user[uids]
    i_emb = final_item[iids]
    preds = (u_emb * i_emb).sum(axis=-1)
    preds = preds + (_lrelu(att_user[u_locs_seq]) * i_emb).sum(axis=-1)
    return preds


def kernel(params, adj_vals, adj_rows, adj_cols, uids, iids, sequences, masks, u_locs_seq, keep_rate):
    return _forward(params, adj_vals, masks, adj_rows, adj_cols, uids, iids, sequences, u_locs_seq)


# speedup vs baseline: 2.0805x; 2.0805x over previous
import functools

import jax
import jax.numpy as jnp
from jax import lax
from jax.experimental import pallas as pl
from jax.experimental.pallas import tpu as pltpu
from jax.experimental.pallas import tpu_sc as plsc

U = 50000; I = 50000; G = 4; D = 128; H = 4; DK = 32
LGNN = 2; LATT = 2; P = 50; B = 4096; NNZ = 500000; LEAKY = 0.5


NC, NS = 2, 16
NNZP = 524288
EB = 4096
EPT = NNZP // NS
NBLK = EPT // EB
NCH = 3
CHUNK = 8384
OUTR = 2 * NCH * CHUNK
CHUNK_PAD = 8448
ZROWS = 48
RPT = 528
SIDE = U * G


def _sc_spmm_body(xs, rowsp, colsp, valsp, out,
                  acc, zbuf, dstb, srcb, valb,
                  sel_src, sel_val, sel_lrow, idx2, lrow2, gbuf, sem):
    c = lax.axis_index("c")
    s = lax.axis_index("s")
    estart = s * EPT
    iota = lax.iota(jnp.int32, 16)

    def zb(r, carry):
        for kk in range(8):
            zbuf[r, pl.ds(kk * 16, 16)] = jnp.zeros((16,), jnp.float32)
        return carry
    lax.fori_loop(0, ZROWS, zb, 0)

    for d in range(2):
        dst_ref = rowsp if d == 0 else colsp
        src_ref = colsp if d == 0 else rowsp

        def g_body(g, carry, d=d, dst_ref=dst_ref, src_ref=src_ref):
            gbase = (1 - d) * SIDE + g * U
            obase = (d * G + g) * OUTR

            def cc_body(cc, carry2):
                lo = (c * NCH + cc) * CHUNK
                for j in range(RPT // ZROWS):
                    pltpu.sync_copy(zbuf, acc.at[pl.ds(s * RPT + j * ZROWS, ZROWS)])
                rem = RPT % ZROWS
                if rem:
                    pltpu.sync_copy(zbuf.at[pl.ds(0, rem)],
                                    acc.at[pl.ds(s * RPT + (RPT // ZROWS) * ZROWS, rem)])
                plsc.subcore_barrier()

                def blk(b, carry3):
                    off = g * NNZP + estart + b * EB
                    pltpu.sync_copy(dst_ref.at[pl.ds(off, EB)], dstb)
                    pltpu.sync_copy(src_ref.at[pl.ds(off, EB)], srcb)
                    pltpu.sync_copy(valsp.at[pl.ds(off, EB)], valb)

                    def grp(v, nsel):
                        r = dstb[pl.ds(v * 16, 16)]
                        m = (r >= lo) & (r < lo + CHUNK)
                        cnt = jnp.sum(m.astype(jnp.int32))
                        plsc.store_compressed(sel_lrow.at[pl.ds(nsel, 16)], r - lo, mask=m)
                        plsc.store_compressed(sel_src.at[pl.ds(nsel, 16)],
                                              srcb[pl.ds(v * 16, 16)] + gbase, mask=m)
                        plsc.store_compressed(sel_val.at[pl.ds(nsel, 16)],
                                              valb[pl.ds(v * 16, 16)], mask=m)
                        return nsel + cnt
                    nsel = lax.fori_loop(0, EB // 16, grp, 0)

                    for kk in range(8):
                        sel_lrow[pl.ds(nsel + kk * 16, 16)] = CHUNK + iota
                        sel_src[pl.ds(nsel + kk * 16, 16)] = gbase + iota + kk * 16
                        sel_val[pl.ds(nsel + kk * 16, 16)] = jnp.zeros((16,), jnp.float32)
                    nb = lax.div(nsel + 127, 128)

                    def batch(j, carry4):
                        jb = j * 128
                        for kk in range(8):
                            idx2[0, pl.ds(kk * 16, 16)] = sel_src[pl.ds(jb + kk * 16, 16)]
                            lrow2[0, pl.ds(kk * 16, 16)] = sel_lrow[pl.ds(jb + kk * 16, 16)]
                        pltpu.async_copy(xs.at[idx2.at[0]], gbuf, sem).wait()

                        def e16(t, carry5):
                            vv = sel_val[pl.ds(jb + t * 16, 16)]
                            for e in range(16):
                                sv = lax.gather(
                                    vv, jnp.full((16, 1), e, jnp.int32),
                                    lax.GatherDimensionNumbers(
                                        offset_dims=(), collapsed_slice_dims=(0,),
                                        start_index_map=(0,)),
                                    (1,),
                                    mode=lax.GatherScatterMode.PROMISE_IN_BOUNDS)
                                row = t * 16 + e
                                for kk in range(8):
                                    gbuf[row, pl.ds(kk * 16, 16)] = (
                                        gbuf[row, pl.ds(kk * 16, 16)] * sv)
                            return carry5
                        lax.fori_loop(0, 8, e16, 0)
                        pltpu.sync_copy(gbuf, acc.at[lrow2.at[0]], add=True)
                        return carry4
                    lax.fori_loop(0, nb, batch, 0)
                    return carry3
                lax.fori_loop(0, NBLK, blk, 0)
                plsc.subcore_barrier()

                @pl.when(s < NS - 1)
                def _():
                    pltpu.sync_copy(acc.at[pl.ds(s * RPT, RPT)],
                                    out.at[pl.ds(obase + lo + s * RPT, RPT)])
                @pl.when(s == NS - 1)
                def _():
                    last = CHUNK - (NS - 1) * RPT
                    pltpu.sync_copy(acc.at[pl.ds((NS - 1) * RPT, last)],
                                    out.at[pl.ds(obase + lo + (NS - 1) * RPT, last)])
                plsc.subcore_barrier()
                return carry2
            lax.fori_loop(0, NCH, cc_body, 0)
            return carry
        lax.fori_loop(0, G, g_body, 0)


@jax.jit
def _sc_spmm_layer(cur_flat, rowsp, colsp, valsp):
    mesh = plsc.VectorSubcoreMesh(core_axis_name="c", subcore_axis_name="s",
                                  num_cores=NC, num_subcores=NS)
    f = pl.kernel(
        _sc_spmm_body,
        out_type=jax.ShapeDtypeStruct((8 * OUTR, D), jnp.float32),
        mesh=mesh,
        compiler_params=pltpu.CompilerParams(needs_layout_passes=False),
        scratch_types=[
            pltpu.VMEM_SHARED((CHUNK_PAD, D), jnp.float32),
            pltpu.VMEM((ZROWS, D), jnp.float32),
            pltpu.VMEM((EB,), jnp.int32),
            pltpu.VMEM((EB,), jnp.int32),
            pltpu.VMEM((EB,), jnp.float32),
            pltpu.VMEM((EB + 144,), jnp.int32),
            pltpu.VMEM((EB + 144,), jnp.float32),
            pltpu.VMEM((EB + 144,), jnp.int32),
            pltpu.VMEM((1, 128), jnp.int32),
            pltpu.VMEM((1, 128), jnp.int32),
            pltpu.VMEM((128, D), jnp.float32),
            pltpu.SemaphoreType.DMA,
        ],
    )
    return f(cur_flat, rowsp, colsp, valsp)


def _lrelu(x):
    return jnp.where(x > 0, x, LEAKY * x)


def _ln(x, g, b, eps=1e-5):
    m = x.mean(-1, keepdims=True)
    v = ((x - m) ** 2).mean(-1, keepdims=True)
    return (x - m) / jnp.sqrt(v + eps) * g + b


def _mhsa(x, Wq, Wk, Wv):
    Bx, S, Dm = x.shape
    q = (x @ Wq.T).reshape(Bx, S, H, DK).transpose(0, 2, 1, 3)
    k = (x @ Wk.T).reshape(Bx, S, H, DK).transpose(0, 2, 1, 3)
    v = (x @ Wv.T).reshape(Bx, S, H, DK).transpose(0, 2, 1, 3)
    scores = q @ k.transpose(0, 1, 3, 2) / DK ** 0.5
    attn = jax.nn.softmax(scores, axis=-1)
    ctx = (attn @ v).transpose(0, 2, 1, 3).reshape(Bx, S, Dm)
    return ctx


def _lstm(x, Wih, Whh, bih, bhh):
    N = x.shape[0]
    def step(carry, xt):
        h, c = carry
        gates = xt @ Wih.T + h @ Whh.T + bih + bhh
        i, f, g, o = jnp.split(gates, 4, axis=-1)
        i = jax.nn.sigmoid(i); f = jax.nn.sigmoid(f)
        g = jnp.tanh(g); o = jax.nn.sigmoid(o)
        c = f * c + i * g
        h = o * jnp.tanh(c)
        return (h, c), h
    h0 = jnp.zeros((N, D), x.dtype); c0 = jnp.zeros((N, D), x.dtype)
    _, hs = jax.lax.scan(step, (h0, c0), x.transpose(1, 0, 2))
    return hs.transpose(1, 0, 2)


def _spmm(vals, rows, cols, x, n_out):
    return jax.ops.segment_sum(vals[:, None] * x[cols], rows, num_segments=n_out)


_ROWS_BLK = 2000


def _epilogue_body(s_ref, cur_ref, acc_ref, cur_out_ref, acc_out_ref):
    s = s_ref[...]
    cur_new = jnp.where(s > 0, s, LEAKY * s) + cur_ref[...]
    cur_out_ref[...] = cur_new
    acc_out_ref[...] = acc_ref[...] + cur_new


def _gnn_epilogue(s_pad, cur, acc):
    grid = (2 * G, U // _ROWS_BLK)
    spec = pl.BlockSpec((1, _ROWS_BLK, D), lambda g, i: (g, i, 0))
    cur_new, acc_new = pl.pallas_call(
        _epilogue_body,
        grid=grid,
        in_specs=[spec, spec, spec],
        out_specs=[spec, spec],
        out_shape=[jax.ShapeDtypeStruct((2 * G, U, D), jnp.float32)] * 2,
    )(s_pad, cur, acc)
    return cur_new, acc_new


def _forward(params, adj_vals, masks, adj_rows, adj_cols, uids, iids, sequences, u_locs_seq):
    cur = jnp.concatenate([params["user_embeds"], params["item_embeds"]], axis=0)
    acc = cur
    pad = NNZP - NNZ
    pad_idx = (jnp.arange(pad, dtype=jnp.int32) * 2053) % U
    rowsp = jnp.concatenate([adj_rows, jnp.broadcast_to(pad_idx, (G, pad))], axis=1)
    colsp = jnp.concatenate([adj_cols, jnp.broadcast_to(pad_idx, (G, pad))], axis=1)
    valsp = jnp.concatenate([adj_vals, jnp.zeros((G, pad), jnp.float32)], axis=1)
    rowsp = rowsp.reshape(G * NNZP)
    colsp = colsp.reshape(G * NNZP)
    valsp = valsp.reshape(G * NNZP)
    for _ in range(LGNN):
        s = _sc_spmm_layer(cur.reshape(2 * SIDE, D), rowsp, colsp, valsp)
        cur, acc = _gnn_epilogue(s.reshape(2 * G, OUTR, D), cur, acc)
    user_stack = acc[:G].transpose(1, 0, 2)
    item_stack = acc[G:].transpose(1, 0, 2)

    user_rnn = _lstm(user_stack, *params["lstm_user"])
    item_rnn = _lstm(item_stack, *params["lstm_item"])
    user_att = _mhsa(_ln(user_rnn, *params["ln_user"]), *params["mhsa_user"])
    item_att = _mhsa(_ln(item_rnn, *params["ln_item"]), *params["mhsa_item"])
    final_user = user_att.mean(axis=1)
    final_item = item_att.mean(axis=1)

    seq_emb = final_item[sequences]
    pos_emb = jnp.broadcast_to(params["pos_embed"][None], (sequences.shape[0], P, D))
    mask_exp = masks[:, :, None]
    att = (_ln(seq_emb, *params["ln_seq"]) + _ln(pos_emb, *params["ln_seq_pos"])) * mask_exp
    for i in range(LATT):
        att_new = _mhsa(_ln(att, *params["ln_seq_layers"][i]), *params["seq_mhsa"][i])
        att = (_lrelu(att_new) + att) * mask_exp
    att_user = att.sum(axis=1)
    u_emb = final_user[uids]
    i_emb = final_item[iids]
    preds = (u_emb * i_emb).sum(axis=-1)
    preds = preds + (_lrelu(att_user[u_locs_seq]) * i_emb).sum(axis=-1)
    return preds


def kernel(params, adj_vals, adj_rows, adj_cols, uids, iids, sequences, masks, u_locs_seq, keep_rate):
    return _forward(params, adj_vals, masks, adj_rows, adj_cols, uids, iids, sequences, u_locs_seq)


# speedup vs baseline: 2.2107x; 1.0626x over previous
import functools

import jax
import jax.numpy as jnp
from jax import lax
from jax.experimental import pallas as pl
from jax.experimental.pallas import tpu as pltpu
from jax.experimental.pallas import tpu_sc as plsc

U = 50000; I = 50000; G = 4; D = 128; H = 4; DK = 32
LGNN = 2; LATT = 2; P = 50; B = 4096; NNZ = 500000; LEAKY = 0.5


NC, NS = 2, 16
NNZP = 524288
EB = 2048
EPT = NNZP // NS
NBLK = EPT // EB
NCH = 3
CHUNK = 8384
OUTR = 2 * NCH * CHUNK
CHUNK_PAD = 8448
ZROWS = 48
RPT = 528
SIDE = U * G


def _sc_spmm_body(xs, rowsp, colsp, valsp, out,
                  acc, zbuf, dstb, srcb, valb,
                  sel_src, sel_val, sel_lrow, idx2, lrow2, gbuf, sem0, sem1):
    c = lax.axis_index("c")
    s = lax.axis_index("s")
    estart = s * EPT
    iota = lax.iota(jnp.int32, 16)

    def zb(r, carry):
        for kk in range(8):
            zbuf[r, pl.ds(kk * 16, 16)] = jnp.zeros((16,), jnp.float32)
        return carry
    lax.fori_loop(0, ZROWS, zb, 0)

    for d in range(2):
        dst_ref = rowsp if d == 0 else colsp
        src_ref = colsp if d == 0 else rowsp

        def g_body(g, carry, d=d, dst_ref=dst_ref, src_ref=src_ref):
            gbase = (1 - d) * SIDE + g * U
            obase = (d * G + g) * OUTR

            def cc_body(cc, carry2):
                lo = (c * NCH + cc) * CHUNK
                for j in range(RPT // ZROWS):
                    pltpu.sync_copy(zbuf, acc.at[pl.ds(s * RPT + j * ZROWS, ZROWS)])
                rem = RPT % ZROWS
                if rem:
                    pltpu.sync_copy(zbuf.at[pl.ds(0, rem)],
                                    acc.at[pl.ds(s * RPT + (RPT // ZROWS) * ZROWS, rem)])
                plsc.subcore_barrier()

                def blk(b, carry3):
                    off = g * NNZP + estart + b * EB
                    pltpu.sync_copy(dst_ref.at[pl.ds(off, EB)], dstb)
                    pltpu.sync_copy(src_ref.at[pl.ds(off, EB)], srcb)
                    pltpu.sync_copy(valsp.at[pl.ds(off, EB)], valb)

                    def grp(v, nsel):
                        r = dstb[pl.ds(v * 16, 16)]
                        m = (r >= lo) & (r < lo + CHUNK)
                        cnt = jnp.sum(m.astype(jnp.int32))
                        plsc.store_compressed(sel_lrow.at[pl.ds(nsel, 16)], r - lo, mask=m)
                        plsc.store_compressed(sel_src.at[pl.ds(nsel, 16)],
                                              srcb[pl.ds(v * 16, 16)] + gbase, mask=m)
                        plsc.store_compressed(sel_val.at[pl.ds(nsel, 16)],
                                              valb[pl.ds(v * 16, 16)], mask=m)
                        return nsel + cnt
                    nsel = lax.fori_loop(0, EB // 16, grp, 0)

                    for kk in range(16):
                        sel_lrow[pl.ds(nsel + kk * 16, 16)] = CHUNK + iota
                        sel_src[pl.ds(nsel + kk * 16, 16)] = gbase + iota + kk * 16
                        sel_val[pl.ds(nsel + kk * 16, 16)] = jnp.zeros((16,), jnp.float32)
                    nb = lax.div(nsel + 127, 128)
                    sems = (sem0, sem1)

                    def prep_fire(j, b):
                        jb = j * 128
                        for kk in range(8):
                            idx2[b, pl.ds(kk * 16, 16)] = sel_src[pl.ds(jb + kk * 16, 16)]
                            lrow2[b, pl.ds(kk * 16, 16)] = sel_lrow[pl.ds(jb + kk * 16, 16)]
                        pltpu.async_copy(xs.at[idx2.at[b]], gbuf.at[b], sems[b])

                    @pl.when(nb > 0)
                    def _():
                        prep_fire(0, 0)

                    def pair(jp, carry4):
                        for b in range(2):
                            j = jp * 2 + b

                            @pl.when(j < nb)
                            def _(j=j, b=b):
                                pltpu.make_async_copy(
                                    xs.at[idx2.at[b]], gbuf.at[b], sems[b]).wait()

                                @pl.when(j + 1 < nb)
                                def _():
                                    prep_fire(j + 1, 1 - b)

                                jb = j * 128

                                def e16(t, carry5):
                                    vv = sel_val[pl.ds(jb + t * 16, 16)]
                                    for e in range(16):
                                        sv = lax.gather(
                                            vv, jnp.full((16, 1), e, jnp.int32),
                                            lax.GatherDimensionNumbers(
                                                offset_dims=(),
                                                collapsed_slice_dims=(0,),
                                                start_index_map=(0,)),
                                            (1,),
                                            mode=lax.GatherScatterMode.PROMISE_IN_BOUNDS)
                                        row = t * 16 + e
                                        for kk in range(8):
                                            gbuf[b, row, pl.ds(kk * 16, 16)] = (
                                                gbuf[b, row, pl.ds(kk * 16, 16)] * sv)
                                    return carry5
                                lax.fori_loop(0, 8, e16, 0)
                                pltpu.sync_copy(gbuf.at[b], acc.at[lrow2.at[b]],
                                                add=True)
                        return carry4
                    lax.fori_loop(0, lax.div(nb + 1, 2), pair, 0)
                    return carry3
                lax.fori_loop(0, NBLK, blk, 0)
                plsc.subcore_barrier()

                @pl.when(s < NS - 1)
                def _():
                    pltpu.sync_copy(acc.at[pl.ds(s * RPT, RPT)],
                                    out.at[pl.ds(obase + lo + s * RPT, RPT)])
                @pl.when(s == NS - 1)
                def _():
                    last = CHUNK - (NS - 1) * RPT
                    pltpu.sync_copy(acc.at[pl.ds((NS - 1) * RPT, last)],
                                    out.at[pl.ds(obase + lo + (NS - 1) * RPT, last)])
                plsc.subcore_barrier()
                return carry2
            lax.fori_loop(0, NCH, cc_body, 0)
            return carry
        lax.fori_loop(0, G, g_body, 0)


@jax.jit
def _sc_spmm_layer(cur_flat, rowsp, colsp, valsp):
    mesh = plsc.VectorSubcoreMesh(core_axis_name="c", subcore_axis_name="s",
                                  num_cores=NC, num_subcores=NS)
    f = pl.kernel(
        _sc_spmm_body,
        out_type=jax.ShapeDtypeStruct((8 * OUTR, D), jnp.float32),
        mesh=mesh,
        compiler_params=pltpu.CompilerParams(needs_layout_passes=False),
        scratch_types=[
            pltpu.VMEM_SHARED((CHUNK_PAD, D), jnp.float32),
            pltpu.VMEM((ZROWS, D), jnp.float32),
            pltpu.VMEM((EB,), jnp.int32),
            pltpu.VMEM((EB,), jnp.int32),
            pltpu.VMEM((EB,), jnp.float32),
            pltpu.VMEM((EB + 272,), jnp.int32),
            pltpu.VMEM((EB + 272,), jnp.float32),
            pltpu.VMEM((EB + 272,), jnp.int32),
            pltpu.VMEM((2, 128), jnp.int32),
            pltpu.VMEM((2, 128), jnp.int32),
            pltpu.VMEM((2, 128, D), jnp.float32),
            pltpu.SemaphoreType.DMA,
            pltpu.SemaphoreType.DMA,
        ],
    )
    return f(cur_flat, rowsp, colsp, valsp)


def _lrelu(x):
    return jnp.where(x > 0, x, LEAKY * x)


def _ln(x, g, b, eps=1e-5):
    m = x.mean(-1, keepdims=True)
    v = ((x - m) ** 2).mean(-1, keepdims=True)
    return (x - m) / jnp.sqrt(v + eps) * g + b


def _mhsa(x, Wq, Wk, Wv):
    Bx, S, Dm = x.shape
    q = (x @ Wq.T).reshape(Bx, S, H, DK).transpose(0, 2, 1, 3)
    k = (x @ Wk.T).reshape(Bx, S, H, DK).transpose(0, 2, 1, 3)
    v = (x @ Wv.T).reshape(Bx, S, H, DK).transpose(0, 2, 1, 3)
    scores = q @ k.transpose(0, 1, 3, 2) / DK ** 0.5
    attn = jax.nn.softmax(scores, axis=-1)
    ctx = (attn @ v).transpose(0, 2, 1, 3).reshape(Bx, S, Dm)
    return ctx


def _lstm(x, Wih, Whh, bih, bhh):
    N = x.shape[0]
    def step(carry, xt):
        h, c = carry
        gates = xt @ Wih.T + h @ Whh.T + bih + bhh
        i, f, g, o = jnp.split(gates, 4, axis=-1)
        i = jax.nn.sigmoid(i); f = jax.nn.sigmoid(f)
        g = jnp.tanh(g); o = jax.nn.sigmoid(o)
        c = f * c + i * g
        h = o * jnp.tanh(c)
        return (h, c), h
    h0 = jnp.zeros((N, D), x.dtype); c0 = jnp.zeros((N, D), x.dtype)
    _, hs = jax.lax.scan(step, (h0, c0), x.transpose(1, 0, 2))
    return hs.transpose(1, 0, 2)


def _spmm(vals, rows, cols, x, n_out):
    return jax.ops.segment_sum(vals[:, None] * x[cols], rows, num_segments=n_out)


_ROWS_BLK = 2000


def _epilogue_body(s_ref, cur_ref, acc_ref, cur_out_ref, acc_out_ref):
    s = s_ref[...]
    cur_new = jnp.where(s > 0, s, LEAKY * s) + cur_ref[...]
    cur_out_ref[...] = cur_new
    acc_out_ref[...] = acc_ref[...] + cur_new


def _gnn_epilogue(s_pad, cur, acc):
    grid = (2 * G, U // _ROWS_BLK)
    spec = pl.BlockSpec((1, _ROWS_BLK, D), lambda g, i: (g, i, 0))
    cur_new, acc_new = pl.pallas_call(
        _epilogue_body,
        grid=grid,
        in_specs=[spec, spec, spec],
        out_specs=[spec, spec],
        out_shape=[jax.ShapeDtypeStruct((2 * G, U, D), jnp.float32)] * 2,
    )(s_pad, cur, acc)
    return cur_new, acc_new


def _forward(params, adj_vals, masks, adj_rows, adj_cols, uids, iids, sequences, u_locs_seq):
    cur = jnp.concatenate([params["user_embeds"], params["item_embeds"]], axis=0)
    acc = cur
    pad = NNZP - NNZ
    pad_idx = (jnp.arange(pad, dtype=jnp.int32) * 2053) % U
    rowsp = jnp.concatenate([adj_rows, jnp.broadcast_to(pad_idx, (G, pad))], axis=1)
    colsp = jnp.concatenate([adj_cols, jnp.broadcast_to(pad_idx, (G, pad))], axis=1)
    valsp = jnp.concatenate([adj_vals, jnp.zeros((G, pad), jnp.float32)], axis=1)
    rowsp = rowsp.reshape(G * NNZP)
    colsp = colsp.reshape(G * NNZP)
    valsp = valsp.reshape(G * NNZP)
    for _ in range(LGNN):
        s = _sc_spmm_layer(cur.reshape(2 * SIDE, D), rowsp, colsp, valsp)
        cur, acc = _gnn_epilogue(s.reshape(2 * G, OUTR, D), cur, acc)
    user_stack = acc[:G].transpose(1, 0, 2)
    item_stack = acc[G:].transpose(1, 0, 2)

    user_rnn = _lstm(user_stack, *params["lstm_user"])
    item_rnn = _lstm(item_stack, *params["lstm_item"])
    user_att = _mhsa(_ln(user_rnn, *params["ln_user"]), *params["mhsa_user"])
    item_att = _mhsa(_ln(item_rnn, *params["ln_item"]), *params["mhsa_item"])
    final_user = user_att.mean(axis=1)
    final_item = item_att.mean(axis=1)

    seq_emb = final_item[sequences]
    pos_emb = jnp.broadcast_to(params["pos_embed"][None], (sequences.shape[0], P, D))
    mask_exp = masks[:, :, None]
    att = (_ln(seq_emb, *params["ln_seq"]) + _ln(pos_emb, *params["ln_seq_pos"])) * mask_exp
    for i in range(LATT):
        att_new = _mhsa(_ln(att, *params["ln_seq_layers"][i]), *params["seq_mhsa"][i])
        att = (_lrelu(att_new) + att) * mask_exp
    att_user = att.sum(axis=1)
    u_emb = final_user[uids]
    i_emb = final_item[iids]
    preds = (u_emb * i_emb).sum(axis=-1)
    preds = preds + (_lrelu(att_user[u_locs_seq]) * i_emb).sum(axis=-1)
    return preds


def kernel(params, adj_vals, adj_rows, adj_cols, uids, iids, sequences, masks, u_locs_seq, keep_rate):
    return _forward(params, adj_vals, masks, adj_rows, adj_cols, uids, iids, sequences, u_locs_seq)


# speedup vs baseline: 2.7020x; 1.2222x over previous
import functools

import jax
import jax.numpy as jnp
from jax import lax
from jax.experimental import pallas as pl
from jax.experimental.pallas import tpu as pltpu
from jax.experimental.pallas import tpu_sc as plsc

U = 50000; I = 50000; G = 4; D = 128; H = 4; DK = 32
LGNN = 2; LATT = 2; P = 50; B = 4096; NNZ = 500000; LEAKY = 0.5


NC, NS = 2, 16
NNZP = 524288
EB = 2048
EPT = NNZP // NS
NBLK = EPT // EB
NCH = 3
CHUNK = 8384
OUTR = 2 * NCH * CHUNK
CHUNK_PAD = 8448
ZROWS = 48
RPT = 528
SIDE = U * G


def _sc_spmm_body(xs, rowsp, colsp, valsp, out,
                  acc, zbuf, dstb, srcb, valb,
                  sel_src, sel_val, sel_lrow, idx2, lrow2, gbuf, sem0, sem1):
    c = lax.axis_index("c")
    s = lax.axis_index("s")
    estart = s * EPT
    iota = lax.iota(jnp.int32, 16)

    def zb(r, carry):
        for kk in range(8):
            zbuf[r, pl.ds(kk * 16, 16)] = jnp.zeros((16,), jnp.float32)
        return carry
    lax.fori_loop(0, ZROWS, zb, 0)

    for d in range(2):
        dst_ref = rowsp if d == 0 else colsp
        src_ref = colsp if d == 0 else rowsp

        def g_body(g, carry, d=d, dst_ref=dst_ref, src_ref=src_ref):
            gbase = (1 - d) * SIDE + g * U
            obase = (d * G + g) * OUTR

            def cc_body(cc, carry2):
                lo = (c * NCH + cc) * CHUNK
                for j in range(RPT // ZROWS):
                    pltpu.sync_copy(zbuf, acc.at[pl.ds(s * RPT + j * ZROWS, ZROWS)])
                rem = RPT % ZROWS
                if rem:
                    pltpu.sync_copy(zbuf.at[pl.ds(0, rem)],
                                    acc.at[pl.ds(s * RPT + (RPT // ZROWS) * ZROWS, rem)])
                plsc.subcore_barrier()

                def blk(b, carry3):
                    off = g * NNZP + estart + b * EB
                    pltpu.sync_copy(dst_ref.at[pl.ds(off, EB)], dstb)
                    pltpu.sync_copy(src_ref.at[pl.ds(off, EB)], srcb)
                    pltpu.sync_copy(valsp.at[pl.ds(off, EB)], valb)

                    def grp(v, nsel):
                        r = dstb[pl.ds(v * 16, 16)]
                        m = (r >= lo) & (r < lo + CHUNK)
                        cnt = jnp.sum(m.astype(jnp.int32))
                        plsc.store_compressed(sel_lrow.at[pl.ds(nsel, 16)], r - lo, mask=m)
                        plsc.store_compressed(sel_src.at[pl.ds(nsel, 16)],
                                              srcb[pl.ds(v * 16, 16)] + gbase, mask=m)
                        plsc.store_compressed(sel_val.at[pl.ds(nsel, 16)],
                                              valb[pl.ds(v * 16, 16)], mask=m)
                        return nsel + cnt
                    nsel = lax.fori_loop(0, EB // 16, grp, 0)

                    for kk in range(16):
                        sel_lrow[pl.ds(nsel + kk * 16, 16)] = CHUNK + iota
                        sel_src[pl.ds(nsel + kk * 16, 16)] = gbase + iota + kk * 16
                        sel_val[pl.ds(nsel + kk * 16, 16)] = jnp.zeros((16,), jnp.float32)
                    nb = lax.div(nsel + 127, 128)
                    sems = (sem0, sem1)

                    def prep_fire(j, b):
                        jb = j * 128
                        for kk in range(8):
                            idx2[b, pl.ds(kk * 16, 16)] = sel_src[pl.ds(jb + kk * 16, 16)]
                            lrow2[b, pl.ds(kk * 16, 16)] = sel_lrow[pl.ds(jb + kk * 16, 16)]
                        pltpu.async_copy(xs.at[idx2.at[b]], gbuf.at[b], sems[b])

                    @pl.when(nb > 0)
                    def _():
                        prep_fire(0, 0)

                    def pair(jp, carry4):
                        for b in range(2):
                            j = jp * 2 + b

                            @pl.when(j < nb)
                            def _(j=j, b=b):
                                pltpu.make_async_copy(
                                    xs.at[idx2.at[b]], gbuf.at[b], sems[b]).wait()

                                @pl.when(j + 1 < nb)
                                def _():
                                    prep_fire(j + 1, 1 - b)

                                jb = j * 128

                                def e16(t, carry5):
                                    vv = sel_val[pl.ds(jb + t * 16, 16)]
                                    for e in range(16):
                                        sv = lax.gather(
                                            vv, jnp.full((16, 1), e, jnp.int32),
                                            lax.GatherDimensionNumbers(
                                                offset_dims=(),
                                                collapsed_slice_dims=(0,),
                                                start_index_map=(0,)),
                                            (1,),
                                            mode=lax.GatherScatterMode.PROMISE_IN_BOUNDS)
                                        row = t * 16 + e
                                        for kk in range(8):
                                            gbuf[b, row, pl.ds(kk * 16, 16)] = (
                                                gbuf[b, row, pl.ds(kk * 16, 16)] * sv)
                                    return carry5
                                lax.fori_loop(0, 8, e16, 0)
                                pltpu.sync_copy(gbuf.at[b], acc.at[lrow2.at[b]],
                                                add=True)
                        return carry4
                    lax.fori_loop(0, lax.div(nb + 1, 2), pair, 0)
                    return carry3
                lax.fori_loop(0, NBLK, blk, 0)
                plsc.subcore_barrier()

                @pl.when(s < NS - 1)
                def _():
                    pltpu.sync_copy(acc.at[pl.ds(s * RPT, RPT)],
                                    out.at[pl.ds(obase + lo + s * RPT, RPT)])
                @pl.when(s == NS - 1)
                def _():
                    last = CHUNK - (NS - 1) * RPT
                    pltpu.sync_copy(acc.at[pl.ds((NS - 1) * RPT, last)],
                                    out.at[pl.ds(obase + lo + (NS - 1) * RPT, last)])
                plsc.subcore_barrier()
                return carry2
            lax.fori_loop(0, NCH, cc_body, 0)
            return carry
        lax.fori_loop(0, G, g_body, 0)


@jax.jit
def _sc_spmm_layer(cur_flat, rowsp, colsp, valsp):
    mesh = plsc.VectorSubcoreMesh(core_axis_name="c", subcore_axis_name="s",
                                  num_cores=NC, num_subcores=NS)
    f = pl.kernel(
        _sc_spmm_body,
        out_type=jax.ShapeDtypeStruct((8 * OUTR, D), jnp.float32),
        mesh=mesh,
        compiler_params=pltpu.CompilerParams(needs_layout_passes=False),
        scratch_types=[
            pltpu.VMEM_SHARED((CHUNK_PAD, D), jnp.float32),
            pltpu.VMEM((ZROWS, D), jnp.float32),
            pltpu.VMEM((EB,), jnp.int32),
            pltpu.VMEM((EB,), jnp.int32),
            pltpu.VMEM((EB,), jnp.float32),
            pltpu.VMEM((EB + 272,), jnp.int32),
            pltpu.VMEM((EB + 272,), jnp.float32),
            pltpu.VMEM((EB + 272,), jnp.int32),
            pltpu.VMEM((2, 128), jnp.int32),
            pltpu.VMEM((2, 128), jnp.int32),
            pltpu.VMEM((2, 128, D), jnp.float32),
            pltpu.SemaphoreType.DMA,
            pltpu.SemaphoreType.DMA,
        ],
    )
    return f(cur_flat, rowsp, colsp, valsp)


def _lrelu(x):
    return jnp.where(x > 0, x, LEAKY * x)


def _ln(x, g, b, eps=1e-5):
    m = x.mean(-1, keepdims=True)
    v = ((x - m) ** 2).mean(-1, keepdims=True)
    return (x - m) / jnp.sqrt(v + eps) * g + b


def _mhsa(x, Wq, Wk, Wv):
    Bx, S, Dm = x.shape
    q = (x @ Wq.T).reshape(Bx, S, H, DK).transpose(0, 2, 1, 3)
    k = (x @ Wk.T).reshape(Bx, S, H, DK).transpose(0, 2, 1, 3)
    v = (x @ Wv.T).reshape(Bx, S, H, DK).transpose(0, 2, 1, 3)
    scores = q @ k.transpose(0, 1, 3, 2) / DK ** 0.5
    attn = jax.nn.softmax(scores, axis=-1)
    ctx = (attn @ v).transpose(0, 2, 1, 3).reshape(Bx, S, Dm)
    return ctx


def _lstm(x, Wih, Whh, bih, bhh):
    N = x.shape[0]
    def step(carry, xt):
        h, c = carry
        gates = xt @ Wih.T + h @ Whh.T + bih + bhh
        i, f, g, o = jnp.split(gates, 4, axis=-1)
        i = jax.nn.sigmoid(i); f = jax.nn.sigmoid(f)
        g = jnp.tanh(g); o = jax.nn.sigmoid(o)
        c = f * c + i * g
        h = o * jnp.tanh(c)
        return (h, c), h
    h0 = jnp.zeros((N, D), x.dtype); c0 = jnp.zeros((N, D), x.dtype)
    _, hs = jax.lax.scan(step, (h0, c0), x.transpose(1, 0, 2))
    return hs.transpose(1, 0, 2)


def _spmm(vals, rows, cols, x, n_out):
    return jax.ops.segment_sum(vals[:, None] * x[cols], rows, num_segments=n_out)


_FB = 400


def _ln_rows(x, g, b, eps=1e-5):
    m = x.mean(-1, keepdims=True)
    v = ((x - m) ** 2).mean(-1, keepdims=True)
    return (x - m) / jnp.sqrt(v + eps) * g + b


def _fused_seq_body(x_ref, wih_ref, whh_ref, bias_ref, lng_ref, lnb_ref,
                    wq_ref, wk_ref, wv_ref, out_ref):
    h = jnp.zeros((_FB, D), jnp.float32)
    c = jnp.zeros((_FB, D), jnp.float32)
    wih = wih_ref[0]; whh = whh_ref[0]; bias = bias_ref[0]
    hs = []
    for g in range(G):
        xt = x_ref[g]
        gates = (jnp.dot(xt, wih, preferred_element_type=jnp.float32)
                 + jnp.dot(h, whh, preferred_element_type=jnp.float32) + bias)
        i = jax.nn.sigmoid(gates[:, :D])
        f = jax.nn.sigmoid(gates[:, D:2 * D])
        gg = jnp.tanh(gates[:, 2 * D:3 * D])
        o = jax.nn.sigmoid(gates[:, 3 * D:])
        c = f * c + i * gg
        h = o * jnp.tanh(c)
        hs.append(h)
    lng = lng_ref[0]; lnb = lnb_ref[0]
    wq = wq_ref[0]; wk = wk_ref[0]; wv = wv_ref[0]
    qs, ks, vs = [], [], []
    for g in range(G):
        y = _ln_rows(hs[g], lng, lnb)
        qs.append(jnp.dot(y, wq, preferred_element_type=jnp.float32))
        ks.append(jnp.dot(y, wk, preferred_element_type=jnp.float32))
        vs.append(jnp.dot(y, wv, preferred_element_type=jnp.float32))
    inv_sqrt_dk = 1.0 / (DK ** 0.5)
    head_out = []
    for hh in range(H):
        sl = slice(hh * DK, (hh + 1) * DK)
        qh = [q[:, sl] for q in qs]
        kh = [k[:, sl] for k in ks]
        vh = [v[:, sl] for v in vs]
        acc_h = None
        for g in range(G):
            sc = [jnp.sum(qh[g] * kh[g2], axis=1, keepdims=True) * inv_sqrt_dk
                  for g2 in range(G)]
            mx = sc[0]
            for g2 in range(1, G):
                mx = jnp.maximum(mx, sc[g2])
            es = [jnp.exp(s0 - mx) for s0 in sc]
            tot = es[0]
            for g2 in range(1, G):
                tot = tot + es[g2]
            ctx = es[0] / tot * vh[0]
            for g2 in range(1, G):
                ctx = ctx + es[g2] / tot * vh[g2]
            acc_h = ctx if acc_h is None else acc_h + ctx
        head_out.append(acc_h)
    out_ref[0] = jnp.concatenate(head_out, axis=1) * (1.0 / G)


def _fused_seq_stage(stack, params):
    wih_u, whh_u, bih_u, bhh_u = params["lstm_user"]
    wih_i, whh_i, bih_i, bhh_i = params["lstm_item"]
    wihs = jnp.stack([wih_u.T, wih_i.T])
    whhs = jnp.stack([whh_u.T, whh_i.T])
    biases = jnp.stack([(bih_u + bhh_u)[None, :], (bih_i + bhh_i)[None, :]])
    lng = jnp.stack([params["ln_user"][0][None, :], params["ln_item"][0][None, :]])
    lnb = jnp.stack([params["ln_user"][1][None, :], params["ln_item"][1][None, :]])
    wq_u, wk_u, wv_u = params["mhsa_user"]
    wq_i, wk_i, wv_i = params["mhsa_item"]
    wqs = jnp.stack([wq_u.T, wq_i.T])
    wks = jnp.stack([wk_u.T, wk_i.T])
    wvs = jnp.stack([wv_u.T, wv_i.T])
    grid = (2, U // _FB)
    xspec = pl.BlockSpec((G, _FB, D), lambda sde, i: (sde, i, 0))
    w3 = pl.BlockSpec((1, D, 4 * D), lambda sde, i: (sde, 0, 0))
    w1 = pl.BlockSpec((1, 1, 4 * D), lambda sde, i: (sde, 0, 0))
    wl = pl.BlockSpec((1, 1, D), lambda sde, i: (sde, 0, 0))
    wd = pl.BlockSpec((1, D, D), lambda sde, i: (sde, 0, 0))
    ospec = pl.BlockSpec((1, _FB, D), lambda sde, i: (sde, i, 0))
    out = pl.pallas_call(
        _fused_seq_body,
        grid=grid,
        in_specs=[xspec, w3, w3, w1, wl, wl, wd, wd, wd],
        out_specs=ospec,
        out_shape=jax.ShapeDtypeStruct((2, U, D), jnp.float32),
    )(stack, wihs, whhs, biases, lng, lnb, wqs, wks, wvs)
    return out[0], out[1]


_ROWS_BLK = 2000


def _epilogue_body(s_ref, cur_ref, acc_ref, cur_out_ref, acc_out_ref):
    s = s_ref[...]
    cur_new = jnp.where(s > 0, s, LEAKY * s) + cur_ref[...]
    cur_out_ref[...] = cur_new
    acc_out_ref[...] = acc_ref[...] + cur_new


def _gnn_epilogue(s_pad, cur, acc):
    grid = (2 * G, U // _ROWS_BLK)
    spec = pl.BlockSpec((1, _ROWS_BLK, D), lambda g, i: (g, i, 0))
    cur_new, acc_new = pl.pallas_call(
        _epilogue_body,
        grid=grid,
        in_specs=[spec, spec, spec],
        out_specs=[spec, spec],
        out_shape=[jax.ShapeDtypeStruct((2 * G, U, D), jnp.float32)] * 2,
    )(s_pad, cur, acc)
    return cur_new, acc_new


def _forward(params, adj_vals, masks, adj_rows, adj_cols, uids, iids, sequences, u_locs_seq):
    cur = jnp.concatenate([params["user_embeds"], params["item_embeds"]], axis=0)
    acc = cur
    pad = NNZP - NNZ
    pad_idx = (jnp.arange(pad, dtype=jnp.int32) * 2053) % U
    rowsp = jnp.concatenate([adj_rows, jnp.broadcast_to(pad_idx, (G, pad))], axis=1)
    colsp = jnp.concatenate([adj_cols, jnp.broadcast_to(pad_idx, (G, pad))], axis=1)
    valsp = jnp.concatenate([adj_vals, jnp.zeros((G, pad), jnp.float32)], axis=1)
    rowsp = rowsp.reshape(G * NNZP)
    colsp = colsp.reshape(G * NNZP)
    valsp = valsp.reshape(G * NNZP)
    for _ in range(LGNN):
        s = _sc_spmm_layer(cur.reshape(2 * SIDE, D), rowsp, colsp, valsp)
        cur, acc = _gnn_epilogue(s.reshape(2 * G, OUTR, D), cur, acc)
    final_user, final_item = _fused_seq_stage(acc, params)

    seq_emb = final_item[sequences]
    pos_emb = jnp.broadcast_to(params["pos_embed"][None], (sequences.shape[0], P, D))
    mask_exp = masks[:, :, None]
    att = (_ln(seq_emb, *params["ln_seq"]) + _ln(pos_emb, *params["ln_seq_pos"])) * mask_exp
    for i in range(LATT):
        att_new = _mhsa(_ln(att, *params["ln_seq_layers"][i]), *params["seq_mhsa"][i])
        att = (_lrelu(att_new) + att) * mask_exp
    att_user = att.sum(axis=1)
    u_emb = final_user[uids]
    i_emb = final_item[iids]
    preds = (u_emb * i_emb).sum(axis=-1)
    preds = preds + (_lrelu(att_user[u_locs_seq]) * i_emb).sum(axis=-1)
    return preds


def kernel(params, adj_vals, adj_rows, adj_cols, uids, iids, sequences, masks, u_locs_seq, keep_rate):
    return _forward(params, adj_vals, masks, adj_rows, adj_cols, uids, iids, sequences, u_locs_seq)


# speedup vs baseline: 2.9042x; 1.0748x over previous
import functools

import jax
import jax.numpy as jnp
from jax import lax
from jax.experimental import pallas as pl
from jax.experimental.pallas import tpu as pltpu
from jax.experimental.pallas import tpu_sc as plsc

U = 50000; I = 50000; G = 4; D = 128; H = 4; DK = 32
LGNN = 2; LATT = 2; P = 50; B = 4096; NNZ = 500000; LEAKY = 0.5


NC, NS = 2, 16
NNZP = 524288
EB = 2048
EPT = NNZP // NS
NBLK = EPT // EB
NCH = 3
CHUNK = 8384
OUTR = 2 * NCH * CHUNK
CHUNK_PAD = 8448
ZROWS = 48
RPT = 528
SIDE = U * G


def _sc_spmm_body(xs, rowsp, colsp, valsp, out,
                  acc, zbuf, dstb, srcb, valb,
                  sel_src, sel_val, sel_lrow, idx2, lrow2, gbuf, sem0, sem1):
    c = lax.axis_index("c")
    s = lax.axis_index("s")
    estart = s * EPT
    iota = lax.iota(jnp.int32, 16)

    def zb(r, carry):
        for kk in range(8):
            zbuf[r, pl.ds(kk * 16, 16)] = jnp.zeros((16,), jnp.float32)
        return carry
    lax.fori_loop(0, ZROWS, zb, 0)

    for d in range(2):
        dst_ref = rowsp if d == 0 else colsp
        src_ref = colsp if d == 0 else rowsp

        def g_body(g, carry, d=d, dst_ref=dst_ref, src_ref=src_ref):
            gbase = (1 - d) * SIDE + g * U
            obase = (d * G + g) * OUTR

            def cc_body(cc, carry2):
                lo = (c * NCH + cc) * CHUNK
                for j in range(RPT // ZROWS):
                    pltpu.sync_copy(zbuf, acc.at[pl.ds(s * RPT + j * ZROWS, ZROWS)])
                rem = RPT % ZROWS
                if rem:
                    pltpu.sync_copy(zbuf.at[pl.ds(0, rem)],
                                    acc.at[pl.ds(s * RPT + (RPT // ZROWS) * ZROWS, rem)])
                plsc.subcore_barrier()

                def blk(b, carry3):
                    off = g * NNZP + estart + b * EB
                    pltpu.sync_copy(dst_ref.at[pl.ds(off, EB)], dstb)
                    pltpu.sync_copy(src_ref.at[pl.ds(off, EB)], srcb)
                    pltpu.sync_copy(valsp.at[pl.ds(off, EB)], valb)

                    def grp(v, nsel):
                        r = dstb[pl.ds(v * 16, 16)]
                        m = (r >= lo) & (r < lo + CHUNK)
                        cnt = jnp.sum(m.astype(jnp.int32))
                        plsc.store_compressed(sel_lrow.at[pl.ds(nsel, 16)], r - lo, mask=m)
                        plsc.store_compressed(sel_src.at[pl.ds(nsel, 16)],
                                              srcb[pl.ds(v * 16, 16)] + gbase, mask=m)
                        plsc.store_compressed(sel_val.at[pl.ds(nsel, 16)],
                                              valb[pl.ds(v * 16, 16)], mask=m)
                        return nsel + cnt
                    nsel = lax.fori_loop(0, EB // 16, grp, 0)

                    for kk in range(16):
                        sel_lrow[pl.ds(nsel + kk * 16, 16)] = CHUNK + iota
                        sel_src[pl.ds(nsel + kk * 16, 16)] = gbase + iota + kk * 16
                        sel_val[pl.ds(nsel + kk * 16, 16)] = jnp.zeros((16,), jnp.float32)
                    nb = lax.div(nsel + 127, 128)
                    sems = (sem0, sem1)

                    def prep_fire(j, b):
                        jb = j * 128
                        for kk in range(8):
                            idx2[b, pl.ds(kk * 16, 16)] = sel_src[pl.ds(jb + kk * 16, 16)]
                            lrow2[b, pl.ds(kk * 16, 16)] = sel_lrow[pl.ds(jb + kk * 16, 16)]
                        pltpu.async_copy(xs.at[idx2.at[b]], gbuf.at[b], sems[b])

                    @pl.when(nb > 0)
                    def _():
                        prep_fire(0, 0)

                    def pair(jp, carry4):
                        for b in range(2):
                            j = jp * 2 + b

                            @pl.when(j < nb)
                            def _(j=j, b=b):
                                pltpu.make_async_copy(
                                    xs.at[idx2.at[b]], gbuf.at[b], sems[b]).wait()

                                @pl.when(j + 1 < nb)
                                def _():
                                    prep_fire(j + 1, 1 - b)

                                jb = j * 128

                                def e16(t, carry5):
                                    vv = sel_val[pl.ds(jb + t * 16, 16)]
                                    for e in range(16):
                                        sv = lax.gather(
                                            vv, jnp.full((16, 1), e, jnp.int32),
                                            lax.GatherDimensionNumbers(
                                                offset_dims=(),
                                                collapsed_slice_dims=(0,),
                                                start_index_map=(0,)),
                                            (1,),
                                            mode=lax.GatherScatterMode.PROMISE_IN_BOUNDS)
                                        row = t * 16 + e
                                        for kk in range(8):
                                            gbuf[b, row, pl.ds(kk * 16, 16)] = (
                                                gbuf[b, row, pl.ds(kk * 16, 16)] * sv)
                                    return carry5
                                lax.fori_loop(0, 8, e16, 0)
                                pltpu.sync_copy(gbuf.at[b], acc.at[lrow2.at[b]],
                                                add=True)
                        return carry4
                    lax.fori_loop(0, lax.div(nb + 1, 2), pair, 0)
                    return carry3
                lax.fori_loop(0, NBLK, blk, 0)
                plsc.subcore_barrier()

                @pl.when(s < NS - 1)
                def _():
                    pltpu.sync_copy(acc.at[pl.ds(s * RPT, RPT)],
                                    out.at[pl.ds(obase + lo + s * RPT, RPT)])
                @pl.when(s == NS - 1)
                def _():
                    last = CHUNK - (NS - 1) * RPT
                    pltpu.sync_copy(acc.at[pl.ds((NS - 1) * RPT, last)],
                                    out.at[pl.ds(obase + lo + (NS - 1) * RPT, last)])
                plsc.subcore_barrier()
                return carry2
            lax.fori_loop(0, NCH, cc_body, 0)
            return carry
        lax.fori_loop(0, G, g_body, 0)


@jax.jit
def _sc_spmm_layer(cur_flat, rowsp, colsp, valsp):
    mesh = plsc.VectorSubcoreMesh(core_axis_name="c", subcore_axis_name="s",
                                  num_cores=NC, num_subcores=NS)
    f = pl.kernel(
        _sc_spmm_body,
        out_type=jax.ShapeDtypeStruct((8 * OUTR, D), jnp.float32),
        mesh=mesh,
        compiler_params=pltpu.CompilerParams(needs_layout_passes=False),
        scratch_types=[
            pltpu.VMEM_SHARED((CHUNK_PAD, D), jnp.float32),
            pltpu.VMEM((ZROWS, D), jnp.float32),
            pltpu.VMEM((EB,), jnp.int32),
            pltpu.VMEM((EB,), jnp.int32),
            pltpu.VMEM((EB,), jnp.float32),
            pltpu.VMEM((EB + 272,), jnp.int32),
            pltpu.VMEM((EB + 272,), jnp.float32),
            pltpu.VMEM((EB + 272,), jnp.int32),
            pltpu.VMEM((2, 128), jnp.int32),
            pltpu.VMEM((2, 128), jnp.int32),
            pltpu.VMEM((2, 128, D), jnp.float32),
            pltpu.SemaphoreType.DMA,
            pltpu.SemaphoreType.DMA,
        ],
    )
    return f(cur_flat, rowsp, colsp, valsp)


def _lrelu(x):
    return jnp.where(x > 0, x, LEAKY * x)


def _ln(x, g, b, eps=1e-5):
    m = x.mean(-1, keepdims=True)
    v = ((x - m) ** 2).mean(-1, keepdims=True)
    return (x - m) / jnp.sqrt(v + eps) * g + b


def _mhsa(x, Wq, Wk, Wv):
    Bx, S, Dm = x.shape
    q = (x @ Wq.T).reshape(Bx, S, H, DK).transpose(0, 2, 1, 3)
    k = (x @ Wk.T).reshape(Bx, S, H, DK).transpose(0, 2, 1, 3)
    v = (x @ Wv.T).reshape(Bx, S, H, DK).transpose(0, 2, 1, 3)
    scores = q @ k.transpose(0, 1, 3, 2) / DK ** 0.5
    attn = jax.nn.softmax(scores, axis=-1)
    ctx = (attn @ v).transpose(0, 2, 1, 3).reshape(Bx, S, Dm)
    return ctx


def _lstm(x, Wih, Whh, bih, bhh):
    N = x.shape[0]
    def step(carry, xt):
        h, c = carry
        gates = xt @ Wih.T + h @ Whh.T + bih + bhh
        i, f, g, o = jnp.split(gates, 4, axis=-1)
        i = jax.nn.sigmoid(i); f = jax.nn.sigmoid(f)
        g = jnp.tanh(g); o = jax.nn.sigmoid(o)
        c = f * c + i * g
        h = o * jnp.tanh(c)
        return (h, c), h
    h0 = jnp.zeros((N, D), x.dtype); c0 = jnp.zeros((N, D), x.dtype)
    _, hs = jax.lax.scan(step, (h0, c0), x.transpose(1, 0, 2))
    return hs.transpose(1, 0, 2)


def _spmm(vals, rows, cols, x, n_out):
    return jax.ops.segment_sum(vals[:, None] * x[cols], rows, num_segments=n_out)


_FB = 400


def _ln_rows(x, g, b, eps=1e-5):
    m = x.mean(-1, keepdims=True)
    v = ((x - m) ** 2).mean(-1, keepdims=True)
    return (x - m) / jnp.sqrt(v + eps) * g + b


def _fused_seq_body(x_ref, wih_ref, whh_ref, bias_ref, lng_ref, lnb_ref,
                    wq_ref, wk_ref, wv_ref, out_ref):
    h = jnp.zeros((_FB, D), jnp.float32)
    c = jnp.zeros((_FB, D), jnp.float32)
    wih = wih_ref[0]; whh = whh_ref[0]; bias = bias_ref[0]
    hs = []
    for g in range(G):
        xt = x_ref[g]
        gates = (jnp.dot(xt, wih, preferred_element_type=jnp.float32)
                 + jnp.dot(h, whh, preferred_element_type=jnp.float32) + bias)
        i = jax.nn.sigmoid(gates[:, :D])
        f = jax.nn.sigmoid(gates[:, D:2 * D])
        gg = jnp.tanh(gates[:, 2 * D:3 * D])
        o = jax.nn.sigmoid(gates[:, 3 * D:])
        c = f * c + i * gg
        h = o * jnp.tanh(c)
        hs.append(h)
    lng = lng_ref[0]; lnb = lnb_ref[0]
    wq = wq_ref[0]; wk = wk_ref[0]; wv = wv_ref[0]
    qs, ks, vs = [], [], []
    for g in range(G):
        y = _ln_rows(hs[g], lng, lnb)
        qs.append(jnp.dot(y, wq, preferred_element_type=jnp.float32))
        ks.append(jnp.dot(y, wk, preferred_element_type=jnp.float32))
        vs.append(jnp.dot(y, wv, preferred_element_type=jnp.float32))
    inv_sqrt_dk = 1.0 / (DK ** 0.5)
    head_out = []
    for hh in range(H):
        sl = slice(hh * DK, (hh + 1) * DK)
        qh = [q[:, sl] for q in qs]
        kh = [k[:, sl] for k in ks]
        vh = [v[:, sl] for v in vs]
        acc_h = None
        for g in range(G):
            sc = [jnp.sum(qh[g] * kh[g2], axis=1, keepdims=True) * inv_sqrt_dk
                  for g2 in range(G)]
            mx = sc[0]
            for g2 in range(1, G):
                mx = jnp.maximum(mx, sc[g2])
            es = [jnp.exp(s0 - mx) for s0 in sc]
            tot = es[0]
            for g2 in range(1, G):
                tot = tot + es[g2]
            ctx = es[0] / tot * vh[0]
            for g2 in range(1, G):
                ctx = ctx + es[g2] / tot * vh[g2]
            acc_h = ctx if acc_h is None else acc_h + ctx
        head_out.append(acc_h)
    out_ref[0] = jnp.concatenate(head_out, axis=1) * (1.0 / G)


def _fused_seq_stage(stack, params):
    wih_u, whh_u, bih_u, bhh_u = params["lstm_user"]
    wih_i, whh_i, bih_i, bhh_i = params["lstm_item"]
    wihs = jnp.stack([wih_u.T, wih_i.T])
    whhs = jnp.stack([whh_u.T, whh_i.T])
    biases = jnp.stack([(bih_u + bhh_u)[None, :], (bih_i + bhh_i)[None, :]])
    lng = jnp.stack([params["ln_user"][0][None, :], params["ln_item"][0][None, :]])
    lnb = jnp.stack([params["ln_user"][1][None, :], params["ln_item"][1][None, :]])
    wq_u, wk_u, wv_u = params["mhsa_user"]
    wq_i, wk_i, wv_i = params["mhsa_item"]
    wqs = jnp.stack([wq_u.T, wq_i.T])
    wks = jnp.stack([wk_u.T, wk_i.T])
    wvs = jnp.stack([wv_u.T, wv_i.T])
    grid = (2, U // _FB)
    xspec = pl.BlockSpec((G, _FB, D), lambda sde, i: (sde, i, 0))
    w3 = pl.BlockSpec((1, D, 4 * D), lambda sde, i: (sde, 0, 0))
    w1 = pl.BlockSpec((1, 1, 4 * D), lambda sde, i: (sde, 0, 0))
    wl = pl.BlockSpec((1, 1, D), lambda sde, i: (sde, 0, 0))
    wd = pl.BlockSpec((1, D, D), lambda sde, i: (sde, 0, 0))
    ospec = pl.BlockSpec((1, _FB, D), lambda sde, i: (sde, i, 0))
    out = pl.pallas_call(
        _fused_seq_body,
        grid=grid,
        in_specs=[xspec, w3, w3, w1, wl, wl, wd, wd, wd],
        out_specs=ospec,
        out_shape=jax.ShapeDtypeStruct((2, U, D), jnp.float32),
    )(stack, wihs, whhs, biases, lng, lnb, wqs, wks, wvs)
    return out[0], out[1]


_SB = 128


def _seq_att_body(x_ref, pos_ref, mask_ref, lns_ref, lnl_ref, w_ref, out_ref):
    inv_sqrt_dk = 1.0 / (DK ** 0.5)
    x = x_ref[...]
    pos = pos_ref[...]
    mask3 = mask_ref[...][:, :, None]
    att = (_ln_rows(x, lns_ref[0], lns_ref[1])
           + _ln_rows(pos[None], lns_ref[2], lns_ref[3])) * mask3
    for l in range(LATT):
        y = _ln_rows(att, lnl_ref[l, 0], lnl_ref[l, 1])
        y2 = y.reshape(_SB * P, D)
        q = jnp.dot(y2, w_ref[l, 0], preferred_element_type=jnp.float32)
        k = jnp.dot(y2, w_ref[l, 1], preferred_element_type=jnp.float32)
        v = jnp.dot(y2, w_ref[l, 2], preferred_element_type=jnp.float32)
        heads = []
        for hh in range(H):
            sl = slice(hh * DK, (hh + 1) * DK)
            qh = q[:, sl].reshape(_SB, P, DK)
            kh = k[:, sl].reshape(_SB, P, DK)
            vh = v[:, sl].reshape(_SB, P, DK)
            sc = lax.dot_general(
                qh, kh, (((2,), (2,)), ((0,), (0,))),
                preferred_element_type=jnp.float32) * inv_sqrt_dk
            mx = jnp.max(sc, axis=-1, keepdims=True)
            e = jnp.exp(sc - mx)
            sm = e / jnp.sum(e, axis=-1, keepdims=True)
            ctx = lax.dot_general(
                sm, vh, (((2,), (1,)), ((0,), (0,))),
                preferred_element_type=jnp.float32)
            heads.append(ctx)
        att_new = jnp.concatenate(heads, axis=2)
        att = (jnp.where(att_new > 0, att_new, LEAKY * att_new) + att) * mask3
    out_ref[...] = att.sum(axis=1)


def _seq_att_stage(seq_emb, masks, params):
    lns = jnp.stack([params["ln_seq"][0], params["ln_seq"][1],
                     params["ln_seq_pos"][0], params["ln_seq_pos"][1]])
    lnl = jnp.stack([jnp.stack([g, b]) for g, b in params["ln_seq_layers"]])
    w = jnp.stack([jnp.stack([wq.T, wk.T, wv.T])
                   for wq, wk, wv in params["seq_mhsa"]])
    grid = (B // _SB,)
    out = pl.pallas_call(
        _seq_att_body,
        grid=grid,
        in_specs=[
            pl.BlockSpec((_SB, P, D), lambda i: (i, 0, 0)),
            pl.BlockSpec((P, D), lambda i: (0, 0)),
            pl.BlockSpec((_SB, P), lambda i: (i, 0)),
            pl.BlockSpec((4, D), lambda i: (0, 0)),
            pl.BlockSpec((LATT, 2, D), lambda i: (0, 0, 0)),
            pl.BlockSpec((LATT, 3, D, D), lambda i: (0, 0, 0, 0)),
        ],
        out_specs=pl.BlockSpec((_SB, D), lambda i: (i, 0)),
        out_shape=jax.ShapeDtypeStruct((B, D), jnp.float32),
    )(seq_emb, params["pos_embed"], masks, lns, lnl, w)
    return out


_ROWS_BLK = 2000


def _epilogue_body(s_ref, cur_ref, acc_ref, cur_out_ref, acc_out_ref):
    s = s_ref[...]
    cur_new = jnp.where(s > 0, s, LEAKY * s) + cur_ref[...]
    cur_out_ref[...] = cur_new
    acc_out_ref[...] = acc_ref[...] + cur_new


def _gnn_epilogue(s_pad, cur, acc):
    grid = (2 * G, U // _ROWS_BLK)
    spec = pl.BlockSpec((1, _ROWS_BLK, D), lambda g, i: (g, i, 0))
    cur_new, acc_new = pl.pallas_call(
        _epilogue_body,
        grid=grid,
        in_specs=[spec, spec, spec],
        out_specs=[spec, spec],
        out_shape=[jax.ShapeDtypeStruct((2 * G, U, D), jnp.float32)] * 2,
    )(s_pad, cur, acc)
    return cur_new, acc_new


def _forward(params, adj_vals, masks, adj_rows, adj_cols, uids, iids, sequences, u_locs_seq):
    cur = jnp.concatenate([params["user_embeds"], params["item_embeds"]], axis=0)
    acc = cur
    pad = NNZP - NNZ
    pad_idx = (jnp.arange(pad, dtype=jnp.int32) * 2053) % U
    rowsp = jnp.concatenate([adj_rows, jnp.broadcast_to(pad_idx, (G, pad))], axis=1)
    colsp = jnp.concatenate([adj_cols, jnp.broadcast_to(pad_idx, (G, pad))], axis=1)
    valsp = jnp.concatenate([adj_vals, jnp.zeros((G, pad), jnp.float32)], axis=1)
    rowsp = rowsp.reshape(G * NNZP)
    colsp = colsp.reshape(G * NNZP)
    valsp = valsp.reshape(G * NNZP)
    for _ in range(LGNN):
        s = _sc_spmm_layer(cur.reshape(2 * SIDE, D), rowsp, colsp, valsp)
        cur, acc = _gnn_epilogue(s.reshape(2 * G, OUTR, D), cur, acc)
    final_user, final_item = _fused_seq_stage(acc, params)

    seq_emb = final_item[sequences]
    att_user = _seq_att_stage(seq_emb, masks, params)
    u_emb = final_user[uids]
    i_emb = final_item[iids]
    preds = (u_emb * i_emb).sum(axis=-1)
    preds = preds + (_lrelu(att_user[u_locs_seq]) * i_emb).sum(axis=-1)
    return preds


def kernel(params, adj_vals, adj_rows, adj_cols, uids, iids, sequences, masks, u_locs_seq, keep_rate):
    return _forward(params, adj_vals, masks, adj_rows, adj_cols, uids, iids, sequences, u_locs_seq)
